# Initial kernel scaffold; baseline (speedup 1.0000x reference)
import jax
import jax.numpy as jnp
from jax.experimental import pallas as pl


def kernel(output, target, anchors):
    raise NotImplementedError("write your pallas kernel here")


# speedup vs baseline: 1.7547x; 1.7547x over previous
import functools

import jax
import jax.numpy as jnp
import numpy as np
from jax.experimental import pallas as pl
from jax.experimental.pallas import tpu as pltpu

_NC = 20
_NA = 5
_NB = 64
_NH = 38
_NW = 38
_MAXB = 50
_THRESH = 0.6
_POS = _NA * _NH * _NW
_PPAD = 7680
_ROWS = _PPAD // 128

_P = np.arange(_PPAD)
_A = np.minimum(_P // (_NH * _NW), _NA - 1)
_S = _P % (_NH * _NW)
_VALID = (_P < _POS)
_COL = ((_S % _NW) * _VALID).astype(np.float32).reshape(_ROWS, 128)
_ROW = ((_S // _NW) * _VALID).astype(np.float32).reshape(_ROWS, 128)
_FIOTA = np.where(_VALID, _P, -1).astype(np.int32).reshape(_ROWS, 128)


def _region_loss_kernel(out_ref, tgt_ref, anc_ref, fio_ref, col_ref, row_ref,
                        awm_ref, ahm_ref, o_ref, corr_ref, mxi_ref, mat_ref):
    f32 = jnp.float32
    x = jax.nn.sigmoid(out_ref[0, 0])
    y = jax.nn.sigmoid(out_ref[0, 1])
    w = out_ref[0, 2]
    h = out_ref[0, 3]
    conf = jax.nn.sigmoid(out_ref[0, 4])
    px = x + col_ref[:]
    py = y + row_ref[:]
    pw = jnp.exp(w) * awm_ref[:]
    ph = jnp.exp(h) * ahm_ref[:]
    pa = pw * ph

    m = out_ref[0, 5]
    for c in range(6, 5 + _NC):
        m = jnp.maximum(m, out_ref[0, c])
    se = jnp.exp(out_ref[0, 5] - m)
    for c in range(6, 5 + _NC):
        se = se + jnp.exp(out_ref[0, c] - m)
    lse = m + jnp.log(se)

    zero = jnp.zeros_like(x)
    corr_ref[:] = zero
    mxi_ref[:] = zero
    mat_ref[:] = zero
    fio = fio_ref[:]

    def gt_body(g, carry):
        txg = tgt_ref[0, 0, 5 * g + 1]

        @pl.when(txg != 0.0)
        def _():
            gx = txg * _NW
            gy = tgt_ref[0, 0, 5 * g + 2] * _NH
            gw = tgt_ref[0, 0, 5 * g + 3] * _NW
            gh = tgt_ref[0, 0, 5 * g + 4] * _NH
            cls = tgt_ref[0, 0, 5 * g].astype(jnp.int32)
            gi = jnp.clip(gx.astype(jnp.int32), 0, _NW - 1)
            gj = jnp.clip(gy.astype(jnp.int32), 0, _NH - 1)
            tx = gx - gi.astype(f32)
            ty = gy - gj.astype(f32)
            ga = gw * gh
            bi = jnp.minimum(anc_ref[0, 0], gw) * jnp.minimum(anc_ref[0, 1], gh)
            bu = anc_ref[0, 0] * anc_ref[0, 1] + ga - bi
            bn = jnp.int32(0)
            for n in range(1, _NA):
                i_n = jnp.minimum(anc_ref[0, 2 * n], gw) * \
                    jnp.minimum(anc_ref[0, 2 * n + 1], gh)
                u_n = anc_ref[0, 2 * n] * anc_ref[0, 2 * n + 1] + ga - i_n
                better = i_n * bu > bi * u_n
                bn = jnp.where(better, jnp.int32(n), bn)
                bi = jnp.where(better, i_n, bi)
                bu = jnp.where(better, u_n, bu)
            awb = anc_ref[0, 2 * bn]
            ahb = anc_ref[0, 2 * bn + 1]
            num = jnp.concatenate(
                [jnp.full((8, 64), gw, f32), jnp.full((8, 64), gh, f32)],
                axis=1)
            den = jnp.concatenate(
                [jnp.full((8, 64), awb, f32), jnp.full((8, 64), ahb, f32)],
                axis=1)
            lg = jnp.log(num / den)
            tw = lg[0, 0]
            th = lg[0, 64]
            p = bn * (_NH * _NW) + gj * _NW + gi
            mask = fio == p
            hw = gw * 0.5
            hh = gh * 0.5
            mnx = jnp.minimum(px - pw * 0.5, gx - hw)
            mxx = jnp.maximum(px + pw * 0.5, gx + hw)
            mny = jnp.minimum(py - ph * 0.5, gy - hh)
            mxy = jnp.maximum(py + ph * 0.5, gy + hh)
            cw = pw + gw - (mxx - mnx)
            ch = ph + gh - (mxy - mny)
            inter = jnp.where((cw <= 0.0) | (ch <= 0.0), 0.0, cw * ch)
            union = pa + ga - inter
            iou = inter / union
            mxi_ref[:] = jnp.maximum(mxi_ref[:], iou)
            d_x = (0.5 - tx) * (2.0 * x - tx - 0.5)
            d_y = (0.5 - ty) * (2.0 * y - ty - 0.5)
            d_w = tw * (tw - 2.0 * w)
            d_h = th * (th - 2.0 * h)
            dc = conf - iou
            logit = out_ref[0, 5 + cls]
            delta = (0.5 * (d_x + d_y + d_w + d_h) + 2.5 * (dc * dc)
                     + (lse - logit))
            corr_ref[:] = corr_ref[:] + jnp.where(mask, delta, 0.0)
            mat_ref[:] = jnp.where(mask, 1.0, mat_ref[:])

        return carry

    jax.lax.fori_loop(0, _MAXB, gt_body, 0)

    bxy = (x - 0.5) ** 2 + (y - 0.5) ** 2 + w * w + h * h
    bg = jnp.where(
        (mxi_ref[:] <= _THRESH) & (mat_ref[:] == 0.0) & (fio >= 0),
        conf * conf, 0.0)
    acc = 0.5 * (bxy + bg) + corr_ref[:]
    o_ref[0, 0, 0] = jnp.sum(acc)


@jax.jit
def kernel(output, target, anchors):
    f32 = jnp.float32
    aw = anchors.reshape(_NA, 2)[:, 0]
    ah = anchors.reshape(_NA, 2)[:, 1]
    vmask = jnp.asarray(_VALID.astype(np.float32))
    awm = (aw[_A] * vmask).reshape(_ROWS, 128)
    ahm = (ah[_A] * vmask).reshape(_ROWS, 128)

    out_t = output.reshape(_NB, _NA, 5 + _NC, _NH * _NW)
    out_t = out_t.transpose(0, 2, 1, 3).reshape(_NB, 5 + _NC, _POS)
    out_t = jnp.pad(out_t, ((0, 0), (0, 0), (0, _PPAD - _POS)))
    out_t = out_t.reshape(_NB, 5 + _NC, _ROWS, 128)

    partials = pl.pallas_call(
        _region_loss_kernel,
        grid=(_NB,),
        in_specs=[
            pl.BlockSpec((1, 5 + _NC, _ROWS, 128), lambda b: (b, 0, 0, 0)),
            pl.BlockSpec((1, 1, 5 * _MAXB), lambda b: (b, 0, 0),
                         memory_space=pltpu.SMEM),
            pl.BlockSpec((1, 2 * _NA), lambda b: (0, 0),
                         memory_space=pltpu.SMEM),
            pl.BlockSpec((_ROWS, 128), lambda b: (0, 0)),
            pl.BlockSpec((_ROWS, 128), lambda b: (0, 0)),
            pl.BlockSpec((_ROWS, 128), lambda b: (0, 0)),
            pl.BlockSpec((_ROWS, 128), lambda b: (0, 0)),
            pl.BlockSpec((_ROWS, 128), lambda b: (0, 0)),
        ],
        out_specs=pl.BlockSpec((1, 1, 1), lambda b: (b, 0, 0),
                               memory_space=pltpu.SMEM),
        out_shape=jax.ShapeDtypeStruct((_NB, 1, 1), f32),
        scratch_shapes=[
            pltpu.VMEM((_ROWS, 128), f32),
            pltpu.VMEM((_ROWS, 128), f32),
            pltpu.VMEM((_ROWS, 128), f32),
        ],
        compiler_params=pltpu.CompilerParams(
            dimension_semantics=("parallel",)),
    )(out_t, target.reshape(_NB, 1, 5 * _MAXB), anchors.reshape(1, 2 * _NA),
      jnp.asarray(_FIOTA), jnp.asarray(_COL), jnp.asarray(_ROW), awm, ahm)
    return jnp.sum(partials)


# speedup vs baseline: 1.9628x; 1.1186x over previous
import functools

import jax
import jax.numpy as jnp
import numpy as np
from jax.experimental import pallas as pl
from jax.experimental.pallas import tpu as pltpu

_NC = 20
_NA = 5
_NB = 64
_NH = 38
_NW = 38
_MAXB = 50
_THRESH = 0.6
_POS = _NA * _NH * _NW
_PPAD = 7680
_ROWS = _PPAD // 128

_P = np.arange(_PPAD)
_A = np.minimum(_P // (_NH * _NW), _NA - 1)
_S = _P % (_NH * _NW)
_VALID = (_P < _POS)
_COL = ((_S % _NW) * _VALID).astype(np.float32).reshape(_ROWS, 128)
_ROW = ((_S // _NW) * _VALID).astype(np.float32).reshape(_ROWS, 128)
_FIOTA = np.where(_VALID, _P, -1).astype(np.int32).reshape(_ROWS, 128)


def _region_loss_kernel(out_ref, tgt_ref, anc_ref, fio_ref, col_ref, row_ref,
                        awm_ref, ahm_ref, o_ref, corr_ref, mxi_ref, mat_ref):
    f32 = jnp.float32
    x = jax.nn.sigmoid(out_ref[0, 0])
    y = jax.nn.sigmoid(out_ref[0, 1])
    w = out_ref[0, 2]
    h = out_ref[0, 3]
    conf = jax.nn.sigmoid(out_ref[0, 4])
    px = x + col_ref[:]
    py = y + row_ref[:]
    pw = jnp.exp(w) * awm_ref[:]
    ph = jnp.exp(h) * ahm_ref[:]
    pa = pw * ph

    m = out_ref[0, 5]
    for c in range(6, 5 + _NC):
        m = jnp.maximum(m, out_ref[0, c])
    se = jnp.exp(out_ref[0, 5] - m)
    for c in range(6, 5 + _NC):
        se = se + jnp.exp(out_ref[0, c] - m)
    lse = m + jnp.log(se)

    zero = jnp.zeros_like(x)
    corr_ref[:] = zero
    mxi_ref[:] = zero
    mat_ref[:] = zero
    fio = fio_ref[:]

    def gt_cond(g):
        return jnp.logical_and(g < _MAXB, tgt_ref[0, 0, 5 * g + 1] != 0.0)

    def gt_body(g):
        if True:
            txg = tgt_ref[0, 0, 5 * g + 1]
            gx = txg * _NW
            gy = tgt_ref[0, 0, 5 * g + 2] * _NH
            gw = tgt_ref[0, 0, 5 * g + 3] * _NW
            gh = tgt_ref[0, 0, 5 * g + 4] * _NH
            cls = tgt_ref[0, 0, 5 * g].astype(jnp.int32)
            gi = jnp.clip(gx.astype(jnp.int32), 0, _NW - 1)
            gj = jnp.clip(gy.astype(jnp.int32), 0, _NH - 1)
            tx = gx - gi.astype(f32)
            ty = gy - gj.astype(f32)
            ga = gw * gh
            bi = jnp.minimum(anc_ref[0, 0], gw) * jnp.minimum(anc_ref[0, 1], gh)
            bu = anc_ref[0, 0] * anc_ref[0, 1] + ga - bi
            bn = jnp.int32(0)
            for n in range(1, _NA):
                i_n = jnp.minimum(anc_ref[0, 2 * n], gw) * \
                    jnp.minimum(anc_ref[0, 2 * n + 1], gh)
                u_n = anc_ref[0, 2 * n] * anc_ref[0, 2 * n + 1] + ga - i_n
                better = i_n * bu > bi * u_n
                bn = jnp.where(better, jnp.int32(n), bn)
                bi = jnp.where(better, i_n, bi)
                bu = jnp.where(better, u_n, bu)
            awb = anc_ref[0, 2 * bn]
            ahb = anc_ref[0, 2 * bn + 1]
            num = jnp.concatenate(
                [jnp.full((8, 64), gw, f32), jnp.full((8, 64), gh, f32)],
                axis=1)
            den = jnp.concatenate(
                [jnp.full((8, 64), awb, f32), jnp.full((8, 64), ahb, f32)],
                axis=1)
            lg = jnp.log(num / den)
            tw = lg[0, 0]
            th = lg[0, 64]
            p = bn * (_NH * _NW) + gj * _NW + gi
            mask = fio == p
            hw = gw * 0.5
            hh = gh * 0.5
            mnx = jnp.minimum(px - pw * 0.5, gx - hw)
            mxx = jnp.maximum(px + pw * 0.5, gx + hw)
            mny = jnp.minimum(py - ph * 0.5, gy - hh)
            mxy = jnp.maximum(py + ph * 0.5, gy + hh)
            cw = pw + gw - (mxx - mnx)
            ch = ph + gh - (mxy - mny)
            inter = jnp.where((cw <= 0.0) | (ch <= 0.0), 0.0, cw * ch)
            union = pa + ga - inter
            iou = inter / union
            mxi_ref[:] = jnp.maximum(mxi_ref[:], iou)
            d_x = (0.5 - tx) * (2.0 * x - tx - 0.5)
            d_y = (0.5 - ty) * (2.0 * y - ty - 0.5)
            d_w = tw * (tw - 2.0 * w)
            d_h = th * (th - 2.0 * h)
            dc = conf - iou
            logit = out_ref[0, 5 + cls]
            delta = (0.5 * (d_x + d_y + d_w + d_h) + 2.5 * (dc * dc)
                     + (lse - logit))
            corr_ref[:] = corr_ref[:] + jnp.where(mask, delta, 0.0)
            mat_ref[:] = jnp.where(mask, 1.0, mat_ref[:])
        return g + 1

    jax.lax.while_loop(gt_cond, gt_body, jnp.int32(0))

    bxy = jnp.where(
        fio >= 0,
        (x - 0.5) ** 2 + (y - 0.5) ** 2 + w * w + h * h, 0.0)
    bg = jnp.where(
        (mxi_ref[:] <= _THRESH) & (mat_ref[:] == 0.0) & (fio >= 0),
        conf * conf, 0.0)
    acc = 0.5 * (bxy + bg) + corr_ref[:]
    o_ref[0, 0, 0] = jnp.sum(acc)


@jax.jit
def kernel(output, target, anchors):
    f32 = jnp.float32
    aw = anchors.reshape(_NA, 2)[:, 0]
    ah = anchors.reshape(_NA, 2)[:, 1]
    vmask = jnp.asarray(_VALID.astype(np.float32))
    awm = (aw[_A] * vmask).reshape(_ROWS, 128)
    ahm = (ah[_A] * vmask).reshape(_ROWS, 128)

    out_t = output.reshape(_NB, _NA, 5 + _NC, _NH * _NW)
    out_t = out_t.transpose(0, 2, 1, 3).reshape(_NB, 5 + _NC, _POS)
    out_t = jnp.pad(out_t, ((0, 0), (0, 0), (0, _PPAD - _POS)))
    out_t = out_t.reshape(_NB, 5 + _NC, _ROWS, 128)

    partials = pl.pallas_call(
        _region_loss_kernel,
        grid=(_NB,),
        in_specs=[
            pl.BlockSpec((1, 5 + _NC, _ROWS, 128), lambda b: (b, 0, 0, 0)),
            pl.BlockSpec((1, 1, 5 * _MAXB), lambda b: (b, 0, 0),
                         memory_space=pltpu.SMEM),
            pl.BlockSpec((1, 2 * _NA), lambda b: (0, 0),
                         memory_space=pltpu.SMEM),
            pl.BlockSpec((_ROWS, 128), lambda b: (0, 0)),
            pl.BlockSpec((_ROWS, 128), lambda b: (0, 0)),
            pl.BlockSpec((_ROWS, 128), lambda b: (0, 0)),
            pl.BlockSpec((_ROWS, 128), lambda b: (0, 0)),
            pl.BlockSpec((_ROWS, 128), lambda b: (0, 0)),
        ],
        out_specs=pl.BlockSpec((1, 1, 1), lambda b: (b, 0, 0),
                               memory_space=pltpu.SMEM),
        out_shape=jax.ShapeDtypeStruct((_NB, 1, 1), f32),
        scratch_shapes=[
            pltpu.VMEM((_ROWS, 128), f32),
            pltpu.VMEM((_ROWS, 128), f32),
            pltpu.VMEM((_ROWS, 128), f32),
        ],
        compiler_params=pltpu.CompilerParams(
            dimension_semantics=("parallel",)),
    )(out_t, target.reshape(_NB, 1, 5 * _MAXB), anchors.reshape(1, 2 * _NA),
      jnp.asarray(_FIOTA), jnp.asarray(_COL), jnp.asarray(_ROW), awm, ahm)
    return jnp.sum(partials)
